# Initial kernel scaffold; baseline (speedup 1.0000x reference)
import jax
import jax.numpy as jnp
from jax import lax
from jax.experimental import pallas as pl
from jax.experimental.pallas import tpu as pltpu
from jax.experimental.pallas import tpu_sc as plsc

N = 10000
D = 128
E = 320000

NC = 2
NS = 16
NW = NC * NS
CH = 128
NCHUNK = -(-E // (NW * CH))
EPT = NCHUNK * CH
EP = EPT * NW
PAD = EP - E
NTR = 16
RPT = N // NS


def _hist_body(col_hbm, ones_hbm, z16_hbm, deg_hbm, cidx, ones_v, deg_sh):
    c = lax.axis_index("c")
    s = lax.axis_index("s")
    wid = s * NC + c
    r0 = s * RPT
    pltpu.sync_copy(z16_hbm.at[pl.ds(r0, RPT)], deg_sh.at[pl.ds(r0, RPT)])
    pltpu.sync_copy(ones_hbm, ones_v)
    plsc.subcore_barrier()

    def chunk(i, carry):
        b = wid * EPT + i * CH
        pltpu.sync_copy(col_hbm.at[pl.ds(b, CH)], cidx)
        pltpu.sync_copy(ones_v, deg_sh.at[cidx], add=True)
        return carry

    lax.fori_loop(0, NCHUNK, chunk, 0)
    plsc.subcore_barrier()
    pltpu.sync_copy(deg_sh.at[pl.ds(r0, RPT)], deg_hbm.at[c, pl.ds(r0, RPT)])


def _spmm_body(row_hbm, col_hbm, hp_hbm, z128_hbm, acc_hbm,
               ridx, cidx, rows_v, sem, acc_sh):
    c = lax.axis_index("c")
    s = lax.axis_index("s")
    wid = s * NC + c
    r0 = s * RPT

    @pl.when(c == 0)
    def _():
        pltpu.sync_copy(hp_hbm.at[pl.ds(r0, RPT)], acc_sh.at[pl.ds(r0, RPT)])

    @pl.when(c != 0)
    def _():
        pltpu.sync_copy(z128_hbm.at[pl.ds(r0, RPT)], acc_sh.at[pl.ds(r0, RPT)])

    plsc.subcore_barrier()

    def chunk(i, carry):
        b = wid * EPT + i * CH
        pltpu.sync_copy(row_hbm.at[pl.ds(b, CH)], ridx)
        pltpu.sync_copy(col_hbm.at[pl.ds(b, CH)], cidx)
        pltpu.async_copy(hp_hbm.at[ridx], rows_v, sem).wait()
        pltpu.sync_copy(rows_v, acc_sh.at[cidx], add=True)
        return carry

    lax.fori_loop(0, NCHUNK, chunk, 0)
    plsc.subcore_barrier()
    pltpu.sync_copy(acc_sh.at[pl.ds(r0, RPT)], acc_hbm.at[c, pl.ds(r0, RPT)])


def _mm_body(x_ref, w_ref, d0_ref, d1_ref, o_ref):
    h = jnp.dot(x_ref[...], w_ref[...], preferred_element_type=jnp.float32)
    deg = d0_ref[0, :, 0:1] + d1_ref[0, :, 0:1] + 1.0
    o_ref[...] = h * lax.rsqrt(deg)


def _comb_body(a0_ref, a1_ref, d0_ref, d1_ref, b_ref, o_ref):
    deg = d0_ref[0, :, 0:1] + d1_ref[0, :, 0:1] + 1.0
    o_ref[...] = lax.rsqrt(deg) * (a0_ref[0] + a1_ref[0]) + b_ref[0]


@jax.jit
def kernel(x, edge_index, W, bias):
    row = edge_index[0]
    col = edge_index[1]
    row_p = jnp.concatenate([row, jnp.zeros((PAD,), jnp.int32)])
    col_p = jnp.concatenate([col, jnp.full((PAD,), N, jnp.int32)])
    z128 = jnp.zeros((N, D), jnp.float32)
    z16 = jnp.zeros((N, 16), jnp.float32)
    ones = jnp.ones((CH, 16), jnp.float32)

    mesh = plsc.VectorSubcoreMesh(
        core_axis_name="c", subcore_axis_name="s",
        num_cores=NC, num_subcores=NS)

    hist = pl.kernel(
        _hist_body,
        out_type=jax.ShapeDtypeStruct((NC, N, 16), jnp.float32),
        mesh=mesh,
        scratch_types=[
            pltpu.VMEM((CH,), jnp.int32),
            pltpu.VMEM((CH, 16), jnp.float32),
            pltpu.VMEM_SHARED((N + NTR, 16), jnp.float32),
        ],
    )
    deg = hist(col_p, ones, z16)

    grid = 10
    bn = N // grid
    hp = pl.pallas_call(
        _mm_body,
        grid=(grid,),
        in_specs=[
            pl.BlockSpec((bn, D), lambda i: (i, 0)),
            pl.BlockSpec((D, D), lambda i: (0, 0)),
            pl.BlockSpec((1, bn, 16), lambda i: (0, i, 0)),
            pl.BlockSpec((1, bn, 16), lambda i: (1, i, 0)),
        ],
        out_specs=pl.BlockSpec((bn, D), lambda i: (i, 0)),
        out_shape=jax.ShapeDtypeStruct((N, D), jnp.float32),
    )(x, W, deg, deg)

    spmm = pl.kernel(
        _spmm_body,
        out_type=jax.ShapeDtypeStruct((NC, N, D), jnp.float32),
        mesh=mesh,
        scratch_types=[
            pltpu.VMEM((CH,), jnp.int32),
            pltpu.VMEM((CH,), jnp.int32),
            pltpu.VMEM((CH, D), jnp.float32),
            pltpu.SemaphoreType.DMA,
            pltpu.VMEM_SHARED((N + NTR, D), jnp.float32),
        ],
    )
    acc = spmm(row_p, col_p, hp, z128)

    out = pl.pallas_call(
        _comb_body,
        grid=(grid,),
        in_specs=[
            pl.BlockSpec((1, bn, D), lambda i: (0, i, 0)),
            pl.BlockSpec((1, bn, D), lambda i: (1, i, 0)),
            pl.BlockSpec((1, bn, 16), lambda i: (0, i, 0)),
            pl.BlockSpec((1, bn, 16), lambda i: (1, i, 0)),
            pl.BlockSpec((1, D), lambda i: (0, 0)),
        ],
        out_specs=pl.BlockSpec((bn, D), lambda i: (i, 0)),
        out_shape=jax.ShapeDtypeStruct((N, D), jnp.float32),
    )(acc, acc, deg, deg, bias.reshape(1, D))
    return out


# speedup vs baseline: 13.6214x; 13.6214x over previous
import jax
import jax.numpy as jnp
from jax import lax
from jax.experimental import pallas as pl
from jax.experimental.pallas import tpu as pltpu
from jax.experimental.pallas import tpu_sc as plsc

N = 10000
D = 128
E = 320000

NC = 2
NS = 16
NW = NC * NS
CH = 128
NCHUNK = -(-E // (NW * CH))
EPT = NCHUNK * CH
EP = EPT * NW
PAD = EP - E
N2 = 10240
RPT = N2 // NS


def _hist_body(col_hbm, ones_hbm, z16_hbm, deg_hbm, cidx, ones_v, deg_sh):
    c = lax.axis_index("c")
    s = lax.axis_index("s")
    wid = s * NC + c
    r0 = s * RPT
    pltpu.sync_copy(z16_hbm.at[pl.ds(r0, RPT)], deg_sh.at[pl.ds(r0, RPT)])
    pltpu.sync_copy(ones_hbm, ones_v)
    plsc.subcore_barrier()

    def chunk(i, carry):
        b = wid * EPT + i * CH
        pltpu.sync_copy(col_hbm.at[pl.ds(b, CH)], cidx)
        pltpu.sync_copy(ones_v, deg_sh.at[cidx], add=True)
        return carry

    lax.fori_loop(0, NCHUNK, chunk, 0)
    plsc.subcore_barrier()
    pltpu.sync_copy(deg_sh.at[pl.ds(r0, RPT)], deg_hbm.at[c, pl.ds(r0, RPT)])


def _spmm_body(row_hbm, col_hbm, hp_hbm, z128_hbm, acc_hbm,
               ridx, cidx, rows_v, sem, acc_sh):
    c = lax.axis_index("c")
    s = lax.axis_index("s")
    wid = s * NC + c
    r0 = s * RPT

    @pl.when(c == 0)
    def _():
        pltpu.sync_copy(hp_hbm.at[pl.ds(r0, RPT)], acc_sh.at[pl.ds(r0, RPT)])

    @pl.when(c != 0)
    def _():
        pltpu.sync_copy(z128_hbm.at[pl.ds(r0, RPT)], acc_sh.at[pl.ds(r0, RPT)])

    plsc.subcore_barrier()

    def chunk(i, carry):
        b = wid * EPT + i * CH
        pltpu.sync_copy(row_hbm.at[pl.ds(b, CH)], ridx)
        pltpu.sync_copy(col_hbm.at[pl.ds(b, CH)], cidx)
        pltpu.async_copy(hp_hbm.at[ridx], rows_v, sem).wait()
        pltpu.sync_copy(rows_v, acc_sh.at[cidx], add=True)
        return carry

    lax.fori_loop(0, NCHUNK, chunk, 0)
    plsc.subcore_barrier()
    pltpu.sync_copy(acc_sh.at[pl.ds(r0, RPT)], acc_hbm.at[c, pl.ds(r0, RPT)])


def _mm_body(x_ref, w_ref, d0_ref, d1_ref, o_ref):
    h = jnp.dot(x_ref[...], w_ref[...], preferred_element_type=jnp.float32)
    deg = d0_ref[0, :, 0:1] + d1_ref[0, :, 0:1] + 1.0
    o_ref[...] = h * lax.rsqrt(deg)


def _comb_body(a0_ref, a1_ref, d0_ref, d1_ref, b_ref, o_ref):
    deg = d0_ref[0, :, 0:1] + d1_ref[0, :, 0:1] + 1.0
    o_ref[...] = lax.rsqrt(deg) * (a0_ref[0] + a1_ref[0]) + b_ref[0]


@jax.jit
def kernel(x, edge_index, W, bias):
    row = edge_index[0]
    col = edge_index[1]
    row_p = jnp.concatenate([row, jnp.zeros((PAD,), jnp.int32)])
    x_p = jnp.concatenate([x, jnp.zeros((N2 - N, D), jnp.float32)])
    col_p = jnp.concatenate([col, jnp.full((PAD,), N, jnp.int32)])
    z128 = jnp.zeros((N2, D), jnp.float32)
    z16 = jnp.zeros((N2, 16), jnp.float32)
    ones = jnp.ones((CH, 16), jnp.float32)

    mesh = plsc.VectorSubcoreMesh(
        core_axis_name="c", subcore_axis_name="s",
        num_cores=NC, num_subcores=NS)

    hist = pl.kernel(
        _hist_body,
        out_type=jax.ShapeDtypeStruct((NC, N2, 16), jnp.float32),
        mesh=mesh,
        scratch_types=[
            pltpu.VMEM((CH,), jnp.int32),
            pltpu.VMEM((CH, 16), jnp.float32),
            pltpu.VMEM_SHARED((N2, 16), jnp.float32),
        ],
    )
    deg = hist(col_p, ones, z16)

    mm_grid = 16
    bm = N2 // mm_grid
    hp = pl.pallas_call(
        _mm_body,
        grid=(mm_grid,),
        in_specs=[
            pl.BlockSpec((bm, D), lambda i: (i, 0)),
            pl.BlockSpec((D, D), lambda i: (0, 0)),
            pl.BlockSpec((1, bm, 16), lambda i: (0, i, 0)),
            pl.BlockSpec((1, bm, 16), lambda i: (1, i, 0)),
        ],
        out_specs=pl.BlockSpec((bm, D), lambda i: (i, 0)),
        out_shape=jax.ShapeDtypeStruct((N2, D), jnp.float32),
    )(x_p, W, deg, deg)

    spmm = pl.kernel(
        _spmm_body,
        out_type=jax.ShapeDtypeStruct((NC, N2, D), jnp.float32),
        mesh=mesh,
        scratch_types=[
            pltpu.VMEM((CH,), jnp.int32),
            pltpu.VMEM((CH,), jnp.int32),
            pltpu.VMEM((CH, D), jnp.float32),
            pltpu.SemaphoreType.DMA,
            pltpu.VMEM_SHARED((N2, D), jnp.float32),
        ],
    )
    acc = spmm(row_p, col_p, hp, z128)

    grid = 10
    bn = N // grid
    out = pl.pallas_call(
        _comb_body,
        grid=(grid,),
        in_specs=[
            pl.BlockSpec((1, bn, D), lambda i: (0, i, 0)),
            pl.BlockSpec((1, bn, D), lambda i: (1, i, 0)),
            pl.BlockSpec((1, bn, 16), lambda i: (0, i, 0)),
            pl.BlockSpec((1, bn, 16), lambda i: (1, i, 0)),
            pl.BlockSpec((1, D), lambda i: (0, 0)),
        ],
        out_specs=pl.BlockSpec((bn, D), lambda i: (i, 0)),
        out_shape=jax.ShapeDtypeStruct((N, D), jnp.float32),
    )(acc, acc, deg, deg, bias.reshape(1, D))
    return out
